# baseline (speedup 1.0000x reference)
import functools
import math

import jax
import jax.numpy as jnp
from jax import lax
from jax.experimental import pallas as pl
from jax.experimental.pallas import tpu as pltpu
from jax.experimental.pallas import tpu_sc as plsc


def _pos_encoding(max_len, embed_dim):
    idx = jnp.arange(0, embed_dim, 2, dtype=jnp.float32)
    pos = jnp.arange(0, max_len, dtype=jnp.float32)[:, None]
    div_term = jnp.exp(-idx / embed_dim * math.log(10000.0))
    ang = pos * div_term
    pe = jnp.zeros((max_len, embed_dim), dtype=jnp.float32)
    pe = pe.at[:, 0::2].set(jnp.sin(ang))
    pe = pe.at[:, 1::2].set(jnp.cos(ang))
    return pe


@functools.lru_cache(maxsize=None)
def _build_sc_kernel(B, L, V, D):
    info = plsc.get_sparse_core_info()
    NC, NS = info.num_cores, info.num_subcores
    NW = NC * NS
    assert B % NW == 0
    SPW = B // NW
    scale = math.sqrt(D)

    chunks = []
    off = 0
    while off < L:
        n = min(128, L - off)
        chunks.append((off, n))
        off += n

    mesh = plsc.VectorSubcoreMesh(core_axis_name="c", subcore_axis_name="s")

    @functools.partial(
        pl.kernel,
        out_type=jax.ShapeDtypeStruct((B * L, D), jnp.float32),
        mesh=mesh,
        scratch_types=[
            pltpu.VMEM((L,), jnp.int32),
            pltpu.VMEM((L, D), jnp.float32),
            pltpu.VMEM((L, D), jnp.float32),
            pltpu.SemaphoreType.DMA,
        ],
        compiler_params=pltpu.CompilerParams(use_tc_tiling_on_sc=False),
    )
    def sc_embed(x_hbm, pe_hbm, w_hbm, out_hbm, idx_v, pe_v, rows_v, sem):
        wid = lax.axis_index("s") * NC + lax.axis_index("c")
        pltpu.sync_copy(pe_hbm, pe_v)

        @pl.loop(0, SPW)
        def _seq(i):
            base = (wid * SPW + i) * L
            pltpu.sync_copy(x_hbm.at[pl.ds(base, L)], idx_v)
            cps = [
                pltpu.async_copy(
                    w_hbm.at[idx_v.at[pl.ds(o, n)]], rows_v.at[pl.ds(o, n)], sem
                )
                for o, n in chunks
            ]
            for cp in cps:
                cp.wait()

            @pl.loop(0, L)
            def _row(r):
                for j in range(D // 16):
                    sl = pl.ds(j * 16, 16)
                    rows_v[r, sl] = rows_v[r, sl] * scale + pe_v[r, sl]

            pltpu.sync_copy(rows_v, out_hbm.at[pl.ds(base, L)])

    return sc_embed


def kernel(x, W):
    B, L = x.shape
    V, D = W.shape
    pe = _pos_encoding(L, D)
    sc_embed = _build_sc_kernel(B, L, V, D)
    out = sc_embed(x.reshape(B * L), pe, W)
    return out.reshape(B, L, D)


# speedup vs baseline: 1.2274x; 1.2274x over previous
import functools
import math

import jax
import jax.numpy as jnp
from jax import lax
from jax.experimental import pallas as pl
from jax.experimental.pallas import tpu as pltpu
from jax.experimental.pallas import tpu_sc as plsc


def _pos_encoding(max_len, embed_dim):
    idx = jnp.arange(0, embed_dim, 2, dtype=jnp.float32)
    pos = jnp.arange(0, max_len, dtype=jnp.float32)[:, None]
    div_term = jnp.exp(-idx / embed_dim * math.log(10000.0))
    ang = pos * div_term
    pe = jnp.zeros((max_len, embed_dim), dtype=jnp.float32)
    pe = pe.at[:, 0::2].set(jnp.sin(ang))
    pe = pe.at[:, 1::2].set(jnp.cos(ang))
    return pe


@functools.lru_cache(maxsize=None)
def _build_sc_kernel(B, L, V, D):
    info = plsc.get_sparse_core_info()
    NC, NS = info.num_cores, info.num_subcores
    NW = NC * NS
    assert B % NW == 0
    SPW = B // NW
    C = 4
    assert SPW % C == 0
    G = SPW // C
    assert G >= 2
    CL = C * L
    scale = math.sqrt(D)

    pieces = []
    off = 0
    while off < CL:
        n = min(128, CL - off)
        pieces.append((off, n))
        off += n

    mesh = plsc.VectorSubcoreMesh(core_axis_name="c", subcore_axis_name="s")

    @functools.partial(
        pl.kernel,
        out_type=jax.ShapeDtypeStruct((B * L, D), jnp.float32),
        mesh=mesh,
        scratch_types=[
            pltpu.VMEM((CL,), jnp.int32),
            pltpu.VMEM((CL,), jnp.int32),
            pltpu.VMEM((L, D), jnp.float32),
            pltpu.VMEM((CL, D), jnp.float32),
            pltpu.VMEM((CL, D), jnp.float32),
            pltpu.SemaphoreType.DMA,
            pltpu.SemaphoreType.DMA,
            pltpu.SemaphoreType.DMA,
            pltpu.SemaphoreType.DMA,
            pltpu.SemaphoreType.DMA,
            pltpu.SemaphoreType.DMA,
        ],
        compiler_params=pltpu.CompilerParams(use_tc_tiling_on_sc=False),
    )
    def sc_embed(x_hbm, pe_hbm, w_hbm, out_hbm,
                 i0, i1, pe_v, r0, r1, sg0, sg1, so0, so1, si0, si1):
        wid = lax.axis_index("s") * NC + lax.axis_index("c")
        base = wid * SPW * L
        ibuf, rbuf = [i0, i1], [r0, r1]
        sg, so, si = [sg0, sg1], [so0, so1], [si0, si1]

        pltpu.sync_copy(pe_hbm, pe_v)

        def idx_copy(g, p):
            off = base + g * CL
            return pltpu.async_copy(x_hbm.at[pl.ds(off, CL)], ibuf[p], si[p])

        def gather(p):
            return [
                pltpu.async_copy(
                    w_hbm.at[ibuf[p].at[pl.ds(o, n)]],
                    rbuf[p].at[pl.ds(o, n)],
                    sg[p],
                )
                for o, n in pieces
            ]

        def out_write(g, p):
            off = base + g * CL
            return pltpu.async_copy(rbuf[p], out_hbm.at[pl.ds(off, CL)], so[p])

        def compute(p):
            rb = rbuf[p]

            @pl.loop(0, L)
            def _pos(r):
                pes = [pe_v[r, pl.ds(16 * j, 16)] for j in range(D // 16)]
                for c in range(C):
                    for j in range(D // 16):
                        sl = pl.ds(16 * j, 16)
                        rb[c * L + r, sl] = rb[c * L + r, sl] * scale + pes[j]

        idx_copy(0, 0).wait()
        gd = {0: gather(0)}
        di = {1: idx_copy(1, 1)}
        od = {}
        for g in range(G):
            p = g % 2
            for d in gd[g]:
                d.wait()
            if g + 2 < G:
                di[g + 2] = idx_copy(g + 2, p)
            if g + 1 < G:
                di[g + 1].wait()
                if g >= 1:
                    od[g - 1].wait()
                gd[g + 1] = gather(1 - p)
            compute(p)
            od[g] = out_write(g, p)
        od[G - 2].wait()
        od[G - 1].wait()

    return sc_embed


def kernel(x, W):
    B, L = x.shape
    V, D = W.shape
    pe = _pos_encoding(L, D)
    sc_embed = _build_sc_kernel(B, L, V, D)
    out = sc_embed(x.reshape(B * L), pe, W)
    return out.reshape(B, L, D)
